# baseline (speedup 1.0000x reference)
import functools

import jax
import jax.numpy as jnp
from jax import lax
from jax.experimental import pallas as pl
from jax.experimental.pallas import tpu as pltpu
from jax.experimental.pallas import tpu_sc as plsc

N = 10000
E = 320000
D_IN = 128
DIM = 64
D_EDGE = 16
CUTOFF = 8.0
N_BLOCKS = 3
N_GRAPHS = 100

NPAD = 10240
LOG2 = 0.6931471805599453

NC = 2
NS = 16
CHUNK = 80
EDGES_PER_WORKER = E // (NC * NS)
NCHUNK = EDGES_PER_WORKER // CHUNK
ROWS_PER_SUB = NPAD // NS


LOG2E = 1.4426950408889634


def _ssp(v):
    t = lax.exp2(jnp.abs(v) * -LOG2E)
    return jnp.maximum(v, 0.0) + jnp.log2(1.0 + t) * LOG2 - LOG2


BE = 2000
E2 = E // 2
NE = E2 // BE


def _we_body(eaA_ref, eaB_ref, ewA_ref, ewB_ref,
             w1t_ref, b1_ref, w2t_ref, b2_ref, o0_ref, o1_ref, o2_ref):
    outs = (o0_ref, o1_ref, o2_ref)
    for half, (ea_ref, ew_ref) in enumerate(((eaA_ref, ewA_ref),
                                             (eaB_ref, ewB_ref))):
        ea = ea_ref[...]
        c = 0.5 * (jnp.cos(ew_ref[...] * (jnp.pi / CUTOFF)) + 1.0)
        for k in range(N_BLOCKS):
            h = _ssp(jnp.dot(ea, w1t_ref[k],
                             preferred_element_type=jnp.float32) + b1_ref[k])
            we = (jnp.dot(h, w2t_ref[k],
                          preferred_element_type=jnp.float32) + b2_ref[k]) * c
            outs[k][:, half * DIM:(half + 1) * DIM] = we


def _we_call(edge_attr, ew2d, w1t, b1, w2t, b2):
    full = lambda *shape: pl.BlockSpec(shape, lambda e: (0,) * len(shape))
    return pl.pallas_call(
        _we_body,
        grid=(NE,),
        in_specs=[
            pl.BlockSpec((BE, D_EDGE), lambda e: (e, 0)),
            pl.BlockSpec((BE, D_EDGE), lambda e: (e + NE, 0)),
            pl.BlockSpec((BE, 1), lambda e: (e, 0)),
            pl.BlockSpec((BE, 1), lambda e: (e + NE, 0)),
            full(N_BLOCKS, D_EDGE, DIM),
            full(N_BLOCKS, 1, DIM),
            full(N_BLOCKS, DIM, DIM),
            full(N_BLOCKS, 1, DIM),
        ],
        out_specs=[pl.BlockSpec((BE, 2 * DIM), lambda e: (e, 0))] * N_BLOCKS,
        out_shape=[jax.ShapeDtypeStruct((E2, 2 * DIM), jnp.float32)] * N_BLOCKS,
    )(edge_attr, edge_attr, ew2d, ew2d, w1t, b1, w2t, b2)


RN = 1000


def _pre_body(x_ref, wpt_ref, bp_ref, wl1t_ref, out_ref, x1_ref):
    o = jnp.maximum(
        jnp.dot(x_ref[...], wpt_ref[...], preferred_element_type=jnp.float32)
        + bp_ref[...], 0.0)
    out_ref[...] = o
    x1_ref[...] = jnp.dot(o, wl1t_ref[...], preferred_element_type=jnp.float32)


def _pre_call(x, wpt, bp, wl1t):
    grid = (N // RN,)
    return pl.pallas_call(
        _pre_body,
        grid=grid,
        in_specs=[
            pl.BlockSpec((RN, D_IN), lambda i: (i, 0)),
            pl.BlockSpec((D_IN, DIM), lambda i: (0, 0)),
            pl.BlockSpec((1, DIM), lambda i: (0, 0)),
            pl.BlockSpec((DIM, DIM), lambda i: (0, 0)),
        ],
        out_specs=[pl.BlockSpec((RN, DIM), lambda i: (i, 0))] * 2,
        out_shape=[jax.ShapeDtypeStruct((N, DIM), jnp.float32)] * 2,
    )(x, wpt, bp, wl1t)


def _sc_body(x1_hbm, we_hbm, src_hbm, dst_hbm, out_hbm, agg_sp, x1_sp,
             src_v0, dst_v0, rows_v0, we_v0,
             src_v1, dst_v1, rows_v1, we_v1,
             sem_i0, sem_w0, sem_g0, sem_i1, sem_w1, sem_g1):
    c = lax.axis_index("c")
    s = lax.axis_index("s")
    wid = c * NS + s
    ebase = wid * EDGES_PER_WORKER

    bufs = ((src_v0, dst_v0, rows_v0, we_v0, sem_i0, sem_w0, sem_g0),
            (src_v1, dst_v1, rows_v1, we_v1, sem_i1, sem_w1, sem_g1))

    nrow = N // NS
    pltpu.async_copy(x1_hbm.at[pl.ds(s * nrow, nrow)],
                     x1_sp.at[pl.ds(s * nrow, nrow)], sem_g0)

    @plsc.parallel_loop(0, CHUNK, unroll=4)
    def _(i):
        for k in range(DIM // 16):
            rows_v0[i, pl.ds(k * 16, 16)] = jnp.zeros((16,), jnp.float32)
    for j in range(ROWS_PER_SUB // CHUNK):
        pltpu.sync_copy(rows_v0,
                        agg_sp.at[pl.ds(s * ROWS_PER_SUB + j * CHUNK, CHUNK)])
    pltpu.make_async_copy(x1_hbm.at[pl.ds(0, nrow)],
                          x1_sp.at[pl.ds(0, nrow)], sem_g0).wait()
    plsc.subcore_barrier()

    col = c * DIM

    def issue_copies(cid, b):
        src_v, dst_v, _, we_v, sem_i, sem_w, _ = bufs[b]
        base = ebase + cid * CHUNK
        rbase = s * EDGES_PER_WORKER + cid * CHUNK
        pltpu.async_copy(src_hbm.at[pl.ds(base, CHUNK)], src_v, sem_i)
        pltpu.async_copy(dst_hbm.at[pl.ds(base, CHUNK)], dst_v, sem_i)
        pltpu.async_copy(we_hbm.at[pl.ds(rbase, CHUNK), pl.ds(col, DIM)],
                         we_v, sem_w)

    def wait_idx(b):
        src_v, dst_v, _, _, sem_i, _, _ = bufs[b]
        pltpu.make_async_copy(src_hbm.at[pl.ds(0, CHUNK)], src_v, sem_i).wait()
        pltpu.make_async_copy(dst_hbm.at[pl.ds(0, CHUNK)], dst_v, sem_i).wait()

    def issue_gather(b):
        src_v, _, rows_v, _, _, _, sem_g = bufs[b]
        pltpu.async_copy(x1_sp.at[src_v], rows_v, sem_g)

    def finish_chunk(b):
        src_v, dst_v, rows_v, we_v, _, sem_w, sem_g = bufs[b]
        pltpu.make_async_copy(we_hbm.at[pl.ds(0, CHUNK), pl.ds(0, DIM)],
                              we_v, sem_w).wait()
        pltpu.make_async_copy(x1_sp.at[src_v], rows_v, sem_g).wait()

        @plsc.parallel_loop(0, CHUNK, unroll=2)
        def _(i):
            for k in range(DIM // 16):
                sl = pl.ds(k * 16, 16)
                rows_v[i, sl] = rows_v[i, sl] * we_v[i, sl]
        pltpu.sync_copy(rows_v, agg_sp.at[dst_v], add=True)

    issue_copies(0, 0)
    issue_copies(1, 1)
    wait_idx(0)
    issue_gather(0)

    def step(jj, _):
        c0 = 2 * jj
        wait_idx(1)
        issue_gather(1)
        finish_chunk(0)
        issue_copies(c0 + 2, 0)
        wait_idx(0)
        issue_gather(0)
        finish_chunk(1)

        @pl.when(c0 + 3 < NCHUNK)
        def _():
            issue_copies(c0 + 3, 1)
        return 0

    lax.fori_loop(0, NCHUNK // 2, step, 0)
    finish_chunk(0)
    plsc.subcore_barrier()

    pltpu.sync_copy(agg_sp.at[pl.ds(s * ROWS_PER_SUB, ROWS_PER_SUB)],
                    out_hbm.at[c, pl.ds(s * ROWS_PER_SUB, ROWS_PER_SUB)])


@functools.lru_cache(maxsize=1)
def _get_sc_message():
    return pl.kernel(
        _sc_body,
        out_type=jax.ShapeDtypeStruct((NC, NPAD, DIM), jnp.float32),
        mesh=plsc.VectorSubcoreMesh(core_axis_name="c", subcore_axis_name="s",
                                    num_cores=NC, num_subcores=NS),
        scratch_types=[
            pltpu.VMEM_SHARED((NPAD, DIM), jnp.float32),
            pltpu.VMEM_SHARED((N, DIM), jnp.float32),
            pltpu.VMEM((CHUNK,), jnp.int32),
            pltpu.VMEM((CHUNK,), jnp.int32),
            pltpu.VMEM((CHUNK, DIM), jnp.float32),
            pltpu.VMEM((CHUNK, DIM), jnp.float32),
            pltpu.VMEM((CHUNK,), jnp.int32),
            pltpu.VMEM((CHUNK,), jnp.int32),
            pltpu.VMEM((CHUNK, DIM), jnp.float32),
            pltpu.VMEM((CHUNK, DIM), jnp.float32),
            pltpu.SemaphoreType.DMA,
            pltpu.SemaphoreType.DMA,
            pltpu.SemaphoreType.DMA,
            pltpu.SemaphoreType.DMA,
            pltpu.SemaphoreType.DMA,
            pltpu.SemaphoreType.DMA,
        ],
        compiler_params=pltpu.CompilerParams(use_tc_tiling_on_sc=False),
    )


def _sc_message(x1, we_k, src, dst):
    return _get_sc_message()(x1, we_k, src, dst)


def _upd_body(aggp_ref, prev_ref, wl2t_ref, bl2_ref, wbt_ref, bb_ref,
              out_ref, stats_ref, acc):
    i = pl.program_id(0)

    @pl.when(i == 0)
    def _():
        acc[...] = jnp.zeros_like(acc)

    agg = aggp_ref[0] + aggp_ref[1]
    t = _ssp(jnp.dot(agg, wl2t_ref[...], preferred_element_type=jnp.float32)
             + bl2_ref[...])
    h2 = jnp.dot(t, wbt_ref[...], preferred_element_type=jnp.float32) + bb_ref[...]
    o = prev_ref[...] + h2
    out_ref[...] = o
    acc[0:1, :] += jnp.sum(o, axis=0, keepdims=True)
    acc[1:2, :] += jnp.sum(o * o, axis=0, keepdims=True)

    @pl.when(i == pl.num_programs(0) - 1)
    def _():
        stats_ref[...] = acc[...]


def _upd_call(aggp, prev, wl2t, bl2, wbt, bb):
    grid = (N // RN,)
    return pl.pallas_call(
        _upd_body,
        grid=grid,
        in_specs=[
            pl.BlockSpec((NC, RN, DIM), lambda i: (0, i, 0)),
            pl.BlockSpec((RN, DIM), lambda i: (i, 0)),
            pl.BlockSpec((DIM, DIM), lambda i: (0, 0)),
            pl.BlockSpec((1, DIM), lambda i: (0, 0)),
            pl.BlockSpec((DIM, DIM), lambda i: (0, 0)),
            pl.BlockSpec((1, DIM), lambda i: (0, 0)),
        ],
        out_specs=[pl.BlockSpec((RN, DIM), lambda i: (i, 0)),
                   pl.BlockSpec((8, DIM), lambda i: (0, 0))],
        out_shape=[jax.ShapeDtypeStruct((N, DIM), jnp.float32),
                   jax.ShapeDtypeStruct((8, DIM), jnp.float32)],
        scratch_shapes=[pltpu.VMEM((8, DIM), jnp.float32)],
    )(aggp, prev, wl2t, bl2, wbt, bb)


def _norm_body(raw_ref, stats_ref, g_ref, b_ref, wl1t_ref, out_ref, x1_ref):
    mean = stats_ref[0:1, :] * (1.0 / N)
    var = stats_ref[1:2, :] * (1.0 / N) - mean * mean
    inv = lax.rsqrt(var + 1e-5) * g_ref[...]
    o = (raw_ref[...] - mean) * inv + b_ref[...]
    out_ref[...] = o
    x1_ref[...] = jnp.dot(o, wl1t_ref[...], preferred_element_type=jnp.float32)


def _norm_call(raw, stats, g, b, wl1t):
    grid = (N // RN,)
    return pl.pallas_call(
        _norm_body,
        grid=grid,
        in_specs=[
            pl.BlockSpec((RN, DIM), lambda i: (i, 0)),
            pl.BlockSpec((8, DIM), lambda i: (0, 0)),
            pl.BlockSpec((1, DIM), lambda i: (0, 0)),
            pl.BlockSpec((1, DIM), lambda i: (0, 0)),
            pl.BlockSpec((DIM, DIM), lambda i: (0, 0)),
        ],
        out_specs=[pl.BlockSpec((RN, DIM), lambda i: (i, 0))] * 2,
        out_shape=[jax.ShapeDtypeStruct((N, DIM), jnp.float32)] * 2,
    )(raw, stats, g, b, wl1t)


def _head_body(raw_ref, stats_ref, g_ref, b_ref, batch_ref,
               wpt_ref, bp_ref, wot_ref, bo_ref, y_ref, sums, cnts):
    i = pl.program_id(0)

    @pl.when(i == 0)
    def _():
        sums[...] = jnp.zeros_like(sums)
        cnts[...] = jnp.zeros_like(cnts)

    mean = stats_ref[0:1, :] * (1.0 / N)
    var = stats_ref[1:2, :] * (1.0 / N) - mean * mean
    inv = lax.rsqrt(var + 1e-5) * g_ref[...]
    o = (raw_ref[...] - mean) * inv + b_ref[...]

    ids = lax.broadcasted_iota(jnp.int32, (RN, N_GRAPHS), 1)
    onehot = jnp.where(batch_ref[...] == ids, 1.0, 0.0).astype(jnp.float32)
    dn = (((0,), (0,)), ((), ()))
    sums[0:N_GRAPHS, :] += lax.dot_general(
        onehot, o, dn, preferred_element_type=jnp.float32)
    cnts[0:N_GRAPHS, :] += lax.dot_general(
        onehot, jnp.ones_like(o), dn, preferred_element_type=jnp.float32)

    @pl.when(i == pl.num_programs(0) - 1)
    def _():
        pooled = sums[0:N_GRAPHS, :] / jnp.maximum(cnts[0:N_GRAPHS, :], 1.0)
        p = jnp.maximum(
            jnp.dot(pooled, wpt_ref[...], preferred_element_type=jnp.float32)
            + bp_ref[...], 0.0)
        y_ref[...] = (jnp.dot(p, wot_ref[...],
                              preferred_element_type=jnp.float32)
                      + bo_ref[...])


def _head_call(raw, stats, g, b, batch2d, wpt, bp, wot, bo):
    grid = (N // RN,)
    return pl.pallas_call(
        _head_body,
        grid=grid,
        in_specs=[
            pl.BlockSpec((RN, DIM), lambda i: (i, 0)),
            pl.BlockSpec((8, DIM), lambda i: (0, 0)),
            pl.BlockSpec((1, DIM), lambda i: (0, 0)),
            pl.BlockSpec((1, DIM), lambda i: (0, 0)),
            pl.BlockSpec((RN, 1), lambda i: (i, 0)),
            pl.BlockSpec((DIM, DIM), lambda i: (0, 0)),
            pl.BlockSpec((1, DIM), lambda i: (0, 0)),
            pl.BlockSpec((DIM, 1), lambda i: (0, 0)),
            pl.BlockSpec((1, 1), lambda i: (0, 0)),
        ],
        out_specs=pl.BlockSpec((N_GRAPHS, 1), lambda i: (0, 0)),
        out_shape=jax.ShapeDtypeStruct((N_GRAPHS, 1), jnp.float32),
        scratch_shapes=[pltpu.VMEM((104, DIM), jnp.float32),
                        pltpu.VMEM((104, DIM), jnp.float32)],
    )(raw, stats, g, b, batch2d, wpt, bp, wot, bo)


def kernel(x, edge_index, edge_weight, edge_attr, batch,
           W_pre, b_pre, Wm1, bm1, Wm2, bm2, Wl1, Wl2, bl2,
           Wblin, bblin, bn_gamma, bn_beta, W_post, b_post, W_out, b_out):
    src = edge_index[0]
    dst = edge_index[1]
    ew2d = edge_weight.reshape(E, 1)
    batch2d = batch.reshape(N, 1)

    w1t = jnp.transpose(Wm1, (0, 2, 1))
    w2t = jnp.transpose(Wm2, (0, 2, 1))
    we = _we_call(edge_attr, ew2d, w1t, bm1[:, None, :], w2t, bm2[:, None, :])

    out, x1 = _pre_call(x, W_pre.T, b_pre[None, :], Wl1[0].T)

    for k in range(N_BLOCKS):
        aggp = _sc_message(x1, we[k], src, dst)
        raw, stats = _upd_call(aggp, out, Wl2[k].T, bl2[k][None, :],
                               Wblin[k].T, bblin[k][None, :])
        if k < N_BLOCKS - 1:
            out, x1 = _norm_call(raw, stats, bn_gamma[k][None, :],
                                 bn_beta[k][None, :], Wl1[k + 1].T)
        else:
            y = _head_call(raw, stats, bn_gamma[k][None, :],
                           bn_beta[k][None, :], batch2d,
                           W_post.T, b_post[None, :], W_out.T,
                           b_out[None, :])
    return y.reshape(-1)


# speedup vs baseline: 1.0072x; 1.0072x over previous
import functools

import jax
import jax.numpy as jnp
from jax import lax
from jax.experimental import pallas as pl
from jax.experimental.pallas import tpu as pltpu
from jax.experimental.pallas import tpu_sc as plsc

N = 10000
E = 320000
D_IN = 128
DIM = 64
D_EDGE = 16
CUTOFF = 8.0
N_BLOCKS = 3
N_GRAPHS = 100

NPAD = 10240
LOG2 = 0.6931471805599453

NC = 2
NS = 16
CHUNK = 80
EDGES_PER_WORKER = E // (NC * NS)
NCHUNK = EDGES_PER_WORKER // CHUNK
ROWS_PER_SUB = NPAD // NS


LOG2E = 1.4426950408889634


def _ssp(v):
    t = lax.exp2(jnp.abs(v) * -LOG2E)
    return jnp.maximum(v, 0.0) + jnp.log2(1.0 + t) * LOG2 - LOG2


BE = 2000
E2 = E // 2
NE = E2 // BE


def _we_body(eaA_ref, eaB_ref, ewA_ref, ewB_ref,
             w1t_ref, b1_ref, w2t_ref, b2_ref, o0_ref, o1_ref, o2_ref):
    outs = (o0_ref, o1_ref, o2_ref)
    for half, (ea_ref, ew_ref) in enumerate(((eaA_ref, ewA_ref),
                                             (eaB_ref, ewB_ref))):
        ea = ea_ref[...]
        c = 0.5 * (jnp.cos(ew_ref[...] * (jnp.pi / CUTOFF)) + 1.0)
        for k in range(N_BLOCKS):
            h = _ssp(jnp.dot(ea, w1t_ref[k],
                             preferred_element_type=jnp.float32) + b1_ref[k])
            we = (jnp.dot(h, w2t_ref[k],
                          preferred_element_type=jnp.float32) + b2_ref[k]) * c
            outs[k][:, half * DIM:(half + 1) * DIM] = we


def _we_call(edge_attr, ew2d, w1t, b1, w2t, b2):
    full = lambda *shape: pl.BlockSpec(shape, lambda e: (0,) * len(shape))
    return pl.pallas_call(
        _we_body,
        grid=(NE,),
        in_specs=[
            pl.BlockSpec((BE, D_EDGE), lambda e: (e, 0)),
            pl.BlockSpec((BE, D_EDGE), lambda e: (e + NE, 0)),
            pl.BlockSpec((BE, 1), lambda e: (e, 0)),
            pl.BlockSpec((BE, 1), lambda e: (e + NE, 0)),
            full(N_BLOCKS, D_EDGE, DIM),
            full(N_BLOCKS, 1, DIM),
            full(N_BLOCKS, DIM, DIM),
            full(N_BLOCKS, 1, DIM),
        ],
        out_specs=[pl.BlockSpec((BE, 2 * DIM), lambda e: (e, 0))] * N_BLOCKS,
        out_shape=[jax.ShapeDtypeStruct((E2, 2 * DIM), jnp.float32)] * N_BLOCKS,
    )(edge_attr, edge_attr, ew2d, ew2d, w1t, b1, w2t, b2)


RN = 1000


def _pre_body(x_ref, wpt_ref, bp_ref, wl1t_ref, out_ref, x1_ref):
    o = jnp.maximum(
        jnp.dot(x_ref[...], wpt_ref[...], preferred_element_type=jnp.float32)
        + bp_ref[...], 0.0)
    out_ref[...] = o
    x1_ref[...] = jnp.dot(o, wl1t_ref[...], preferred_element_type=jnp.float32)


def _pre_call(x, wpt, bp, wl1t):
    grid = (N // RN,)
    return pl.pallas_call(
        _pre_body,
        grid=grid,
        in_specs=[
            pl.BlockSpec((RN, D_IN), lambda i: (i, 0)),
            pl.BlockSpec((D_IN, DIM), lambda i: (0, 0)),
            pl.BlockSpec((1, DIM), lambda i: (0, 0)),
            pl.BlockSpec((DIM, DIM), lambda i: (0, 0)),
        ],
        out_specs=[pl.BlockSpec((RN, DIM), lambda i: (i, 0))] * 2,
        out_shape=[jax.ShapeDtypeStruct((N, DIM), jnp.float32)] * 2,
    )(x, wpt, bp, wl1t)


def _sc_body(x1_hbm, we_hbm, src_hbm, dst_hbm, out_hbm, agg_sp,
             src_v0, dst_v0, rows_v0, we_v0,
             src_v1, dst_v1, rows_v1, we_v1,
             sem_i0, sem_w0, sem_g0, sem_i1, sem_w1, sem_g1):
    c = lax.axis_index("c")
    s = lax.axis_index("s")
    wid = c * NS + s
    ebase = wid * EDGES_PER_WORKER

    bufs = ((src_v0, dst_v0, rows_v0, we_v0, sem_i0, sem_w0, sem_g0),
            (src_v1, dst_v1, rows_v1, we_v1, sem_i1, sem_w1, sem_g1))

    @plsc.parallel_loop(0, CHUNK, unroll=4)
    def _(i):
        for k in range(DIM // 16):
            rows_v0[i, pl.ds(k * 16, 16)] = jnp.zeros((16,), jnp.float32)
    for j in range(ROWS_PER_SUB // CHUNK):
        pltpu.sync_copy(rows_v0,
                        agg_sp.at[pl.ds(s * ROWS_PER_SUB + j * CHUNK, CHUNK)])
    plsc.subcore_barrier()

    col = c * DIM

    def issue_copies(cid, b):
        src_v, dst_v, _, we_v, sem_i, sem_w, _ = bufs[b]
        base = ebase + cid * CHUNK
        rbase = s * EDGES_PER_WORKER + cid * CHUNK
        pltpu.async_copy(src_hbm.at[pl.ds(base, CHUNK)], src_v, sem_i)
        pltpu.async_copy(dst_hbm.at[pl.ds(base, CHUNK)], dst_v, sem_i)
        pltpu.async_copy(we_hbm.at[pl.ds(rbase, CHUNK), pl.ds(col, DIM)],
                         we_v, sem_w)

    def wait_idx(b):
        src_v, dst_v, _, _, sem_i, _, _ = bufs[b]
        pltpu.make_async_copy(src_hbm.at[pl.ds(0, CHUNK)], src_v, sem_i).wait()
        pltpu.make_async_copy(dst_hbm.at[pl.ds(0, CHUNK)], dst_v, sem_i).wait()

    def issue_gather(b):
        src_v, _, rows_v, _, _, _, sem_g = bufs[b]
        pltpu.async_copy(x1_hbm.at[src_v], rows_v, sem_g)

    def finish_chunk(b):
        src_v, dst_v, rows_v, we_v, _, sem_w, sem_g = bufs[b]
        pltpu.make_async_copy(we_hbm.at[pl.ds(0, CHUNK), pl.ds(0, DIM)],
                              we_v, sem_w).wait()
        pltpu.make_async_copy(x1_hbm.at[src_v], rows_v, sem_g).wait()

        @plsc.parallel_loop(0, CHUNK, unroll=4)
        def _(i):
            for k in range(DIM // 16):
                sl = pl.ds(k * 16, 16)
                rows_v[i, sl] = rows_v[i, sl] * we_v[i, sl]
        pltpu.sync_copy(rows_v, agg_sp.at[dst_v], add=True)

    issue_copies(0, 0)
    issue_copies(1, 1)
    wait_idx(0)
    issue_gather(0)

    def step(jj, _):
        c0 = 2 * jj
        wait_idx(1)
        issue_gather(1)
        finish_chunk(0)
        issue_copies(c0 + 2, 0)
        wait_idx(0)
        issue_gather(0)
        finish_chunk(1)

        @pl.when(c0 + 3 < NCHUNK)
        def _():
            issue_copies(c0 + 3, 1)
        return 0

    lax.fori_loop(0, NCHUNK // 2, step, 0)
    finish_chunk(0)
    plsc.subcore_barrier()

    pltpu.sync_copy(agg_sp.at[pl.ds(s * ROWS_PER_SUB, ROWS_PER_SUB)],
                    out_hbm.at[c, pl.ds(s * ROWS_PER_SUB, ROWS_PER_SUB)])


@functools.lru_cache(maxsize=1)
def _get_sc_message():
    return pl.kernel(
        _sc_body,
        out_type=jax.ShapeDtypeStruct((NC, NPAD, DIM), jnp.float32),
        mesh=plsc.VectorSubcoreMesh(core_axis_name="c", subcore_axis_name="s",
                                    num_cores=NC, num_subcores=NS),
        scratch_types=[
            pltpu.VMEM_SHARED((NPAD, DIM), jnp.float32),
            pltpu.VMEM((CHUNK,), jnp.int32),
            pltpu.VMEM((CHUNK,), jnp.int32),
            pltpu.VMEM((CHUNK, DIM), jnp.float32),
            pltpu.VMEM((CHUNK, DIM), jnp.float32),
            pltpu.VMEM((CHUNK,), jnp.int32),
            pltpu.VMEM((CHUNK,), jnp.int32),
            pltpu.VMEM((CHUNK, DIM), jnp.float32),
            pltpu.VMEM((CHUNK, DIM), jnp.float32),
            pltpu.SemaphoreType.DMA,
            pltpu.SemaphoreType.DMA,
            pltpu.SemaphoreType.DMA,
            pltpu.SemaphoreType.DMA,
            pltpu.SemaphoreType.DMA,
            pltpu.SemaphoreType.DMA,
        ],
        compiler_params=pltpu.CompilerParams(use_tc_tiling_on_sc=False),
    )


def _sc_message(x1, we_k, src, dst):
    return _get_sc_message()(x1, we_k, src, dst)


def _upd_body(aggp_ref, prev_ref, wl2t_ref, bl2_ref, wbt_ref, bb_ref,
              out_ref, stats_ref, acc):
    i = pl.program_id(0)

    @pl.when(i == 0)
    def _():
        acc[...] = jnp.zeros_like(acc)

    agg = aggp_ref[0] + aggp_ref[1]
    t = _ssp(jnp.dot(agg, wl2t_ref[...], preferred_element_type=jnp.float32)
             + bl2_ref[...])
    h2 = jnp.dot(t, wbt_ref[...], preferred_element_type=jnp.float32) + bb_ref[...]
    o = prev_ref[...] + h2
    out_ref[...] = o
    acc[0:1, :] += jnp.sum(o, axis=0, keepdims=True)
    acc[1:2, :] += jnp.sum(o * o, axis=0, keepdims=True)

    @pl.when(i == pl.num_programs(0) - 1)
    def _():
        stats_ref[...] = acc[...]


def _upd_call(aggp, prev, wl2t, bl2, wbt, bb):
    grid = (N // RN,)
    return pl.pallas_call(
        _upd_body,
        grid=grid,
        in_specs=[
            pl.BlockSpec((NC, RN, DIM), lambda i: (0, i, 0)),
            pl.BlockSpec((RN, DIM), lambda i: (i, 0)),
            pl.BlockSpec((DIM, DIM), lambda i: (0, 0)),
            pl.BlockSpec((1, DIM), lambda i: (0, 0)),
            pl.BlockSpec((DIM, DIM), lambda i: (0, 0)),
            pl.BlockSpec((1, DIM), lambda i: (0, 0)),
        ],
        out_specs=[pl.BlockSpec((RN, DIM), lambda i: (i, 0)),
                   pl.BlockSpec((8, DIM), lambda i: (0, 0))],
        out_shape=[jax.ShapeDtypeStruct((N, DIM), jnp.float32),
                   jax.ShapeDtypeStruct((8, DIM), jnp.float32)],
        scratch_shapes=[pltpu.VMEM((8, DIM), jnp.float32)],
    )(aggp, prev, wl2t, bl2, wbt, bb)


def _norm_body(raw_ref, stats_ref, g_ref, b_ref, wl1t_ref, out_ref, x1_ref):
    mean = stats_ref[0:1, :] * (1.0 / N)
    var = stats_ref[1:2, :] * (1.0 / N) - mean * mean
    inv = lax.rsqrt(var + 1e-5) * g_ref[...]
    o = (raw_ref[...] - mean) * inv + b_ref[...]
    out_ref[...] = o
    x1_ref[...] = jnp.dot(o, wl1t_ref[...], preferred_element_type=jnp.float32)


def _norm_call(raw, stats, g, b, wl1t):
    grid = (N // RN,)
    return pl.pallas_call(
        _norm_body,
        grid=grid,
        in_specs=[
            pl.BlockSpec((RN, DIM), lambda i: (i, 0)),
            pl.BlockSpec((8, DIM), lambda i: (0, 0)),
            pl.BlockSpec((1, DIM), lambda i: (0, 0)),
            pl.BlockSpec((1, DIM), lambda i: (0, 0)),
            pl.BlockSpec((DIM, DIM), lambda i: (0, 0)),
        ],
        out_specs=[pl.BlockSpec((RN, DIM), lambda i: (i, 0))] * 2,
        out_shape=[jax.ShapeDtypeStruct((N, DIM), jnp.float32)] * 2,
    )(raw, stats, g, b, wl1t)


def _head_body(raw_ref, stats_ref, g_ref, b_ref, batch_ref,
               wpt_ref, bp_ref, wot_ref, bo_ref, y_ref, sums, cnts):
    i = pl.program_id(0)

    @pl.when(i == 0)
    def _():
        sums[...] = jnp.zeros_like(sums)
        cnts[...] = jnp.zeros_like(cnts)

    mean = stats_ref[0:1, :] * (1.0 / N)
    var = stats_ref[1:2, :] * (1.0 / N) - mean * mean
    inv = lax.rsqrt(var + 1e-5) * g_ref[...]
    o = (raw_ref[...] - mean) * inv + b_ref[...]

    ids = lax.broadcasted_iota(jnp.int32, (RN, N_GRAPHS), 1)
    onehot = jnp.where(batch_ref[...] == ids, 1.0, 0.0).astype(jnp.float32)
    dn = (((0,), (0,)), ((), ()))
    sums[0:N_GRAPHS, :] += lax.dot_general(
        onehot, o, dn, preferred_element_type=jnp.float32)
    cnts[0:N_GRAPHS, :] += lax.dot_general(
        onehot, jnp.ones_like(o), dn, preferred_element_type=jnp.float32)

    @pl.when(i == pl.num_programs(0) - 1)
    def _():
        pooled = sums[0:N_GRAPHS, :] / jnp.maximum(cnts[0:N_GRAPHS, :], 1.0)
        p = jnp.maximum(
            jnp.dot(pooled, wpt_ref[...], preferred_element_type=jnp.float32)
            + bp_ref[...], 0.0)
        y_ref[...] = (jnp.dot(p, wot_ref[...],
                              preferred_element_type=jnp.float32)
                      + bo_ref[...])


def _head_call(raw, stats, g, b, batch2d, wpt, bp, wot, bo):
    grid = (N // RN,)
    return pl.pallas_call(
        _head_body,
        grid=grid,
        in_specs=[
            pl.BlockSpec((RN, DIM), lambda i: (i, 0)),
            pl.BlockSpec((8, DIM), lambda i: (0, 0)),
            pl.BlockSpec((1, DIM), lambda i: (0, 0)),
            pl.BlockSpec((1, DIM), lambda i: (0, 0)),
            pl.BlockSpec((RN, 1), lambda i: (i, 0)),
            pl.BlockSpec((DIM, DIM), lambda i: (0, 0)),
            pl.BlockSpec((1, DIM), lambda i: (0, 0)),
            pl.BlockSpec((DIM, 1), lambda i: (0, 0)),
            pl.BlockSpec((1, 1), lambda i: (0, 0)),
        ],
        out_specs=pl.BlockSpec((N_GRAPHS, 1), lambda i: (0, 0)),
        out_shape=jax.ShapeDtypeStruct((N_GRAPHS, 1), jnp.float32),
        scratch_shapes=[pltpu.VMEM((104, DIM), jnp.float32),
                        pltpu.VMEM((104, DIM), jnp.float32)],
    )(raw, stats, g, b, batch2d, wpt, bp, wot, bo)


def kernel(x, edge_index, edge_weight, edge_attr, batch,
           W_pre, b_pre, Wm1, bm1, Wm2, bm2, Wl1, Wl2, bl2,
           Wblin, bblin, bn_gamma, bn_beta, W_post, b_post, W_out, b_out):
    src = edge_index[0]
    dst = edge_index[1]
    ew2d = edge_weight.reshape(E, 1)
    batch2d = batch.reshape(N, 1)

    w1t = jnp.transpose(Wm1, (0, 2, 1))
    w2t = jnp.transpose(Wm2, (0, 2, 1))
    we = _we_call(edge_attr, ew2d, w1t, bm1[:, None, :], w2t, bm2[:, None, :])

    out, x1 = _pre_call(x, W_pre.T, b_pre[None, :], Wl1[0].T)

    for k in range(N_BLOCKS):
        aggp = _sc_message(x1, we[k], src, dst)
        raw, stats = _upd_call(aggp, out, Wl2[k].T, bl2[k][None, :],
                               Wblin[k].T, bblin[k][None, :])
        if k < N_BLOCKS - 1:
            out, x1 = _norm_call(raw, stats, bn_gamma[k][None, :],
                                 bn_beta[k][None, :], Wl1[k + 1].T)
        else:
            y = _head_call(raw, stats, bn_gamma[k][None, :],
                           bn_beta[k][None, :], batch2d,
                           W_post.T, b_post[None, :], W_out.T,
                           b_out[None, :])
    return y.reshape(-1)
